# baseline (speedup 1.0000x reference)
import dataclasses
import functools

import jax
import jax.numpy as jnp
from jax import lax
from jax.experimental import pallas as pl
from jax.experimental.pallas import tpu as pltpu
from jax.experimental.pallas import tpu_sc as plsc

NUM_USERS = 5000
NUM_ITEMS = 5000
N = NUM_USERS + NUM_ITEMS
E = 320000
D = 128
N_LAYERS = 3
G = 1000
GS = 8

NC = 2
NS = 16
NW = NC * NS
L = 16

CH = 112
NCH = 90
EW = NCH * CH
NP = 10240
ROWS_PER_TILE = NP // NS

GP = NW * 32
MCH = 128
GCH = 2

_mesh = plsc.VectorSubcoreMesh(
    core_axis_name="c", subcore_axis_name="s", num_cores=NC, num_subcores=NS
)

_sc_params = pltpu.CompilerParams()
if "needs_layout_passes" in pltpu.CompilerParams.__dataclass_fields__:
  _sc_params = dataclasses.replace(_sc_params, needs_layout_passes=False)


def _make_spmm():
  rows_t = [pltpu.VMEM((CH, D), jnp.float32) for _ in range(3)]
  ed_t = [pltpu.VMEM((3, CH), jnp.int32) for _ in range(6)]
  sems = [pltpu.SemaphoreType.DMA for _ in range(12)]

  @functools.partial(
      pl.kernel,
      out_type=jax.ShapeDtypeStruct((NC, NP, D), jnp.float32),
      mesh=_mesh,
      scratch_types=(
          rows_t + ed_t + [pltpu.VMEM_SHARED((NP, D), jnp.float32)] + sems
      ),
      compiler_params=_sc_params,
  )
  def spmm(x_hbm, edata_h, out_h, *scr):
    rows = list(scr[0:3])
    ed = list(scr[3:9])
    acc_sh = scr[9]
    gsem = list(scr[10:13])
    ssem = list(scr[13:16])
    esem = list(scr[16:22])

    ci = lax.axis_index("c")
    s = lax.axis_index("s")
    wid = ci * NS + s

    def e_issue(c, j8):
      pltpu.async_copy(edata_h.at[wid, c], ed[j8], esem[j8])

    def e_wait(c, j8):
      pltpu.make_async_copy(edata_h.at[wid, c], ed[j8], esem[j8]).wait()

    def g_issue(j8, j4):
      pltpu.async_copy(x_hbm.at[ed[j8].at[0]], rows[j4], gsem[j4])

    def g_wait(j8, j4):
      pltpu.make_async_copy(x_hbm.at[ed[j8].at[0]], rows[j4], gsem[j4]).wait()

    def s_issue(j8, j4):
      pltpu.async_copy(rows[j4], acc_sh.at[ed[j8].at[1]], ssem[j4], add=True)

    def s_wait(j8, j4):
      pltpu.make_async_copy(rows[j4], acc_sh.at[ed[j8].at[1]], ssem[j4]).wait()

    def scale(j8, j4):
      rows_v = rows[j4]
      ed_v = ed[j8]

      @pl.loop(0, CH, step=L)
      def _(e0):
        v16 = plsc.bitcast(ed_v[2, pl.ds(e0, L)], jnp.float32)
        for i in range(L):
          bv = jnp.full((L,), v16[i], jnp.float32)
          for j in range(D // L):
            sl = pl.ds(j * L, L)
            rows_v[e0 + i, sl] = rows_v[e0 + i, sl] * bv

    for c in range(4):
      e_issue(c, c)

    zv = jnp.zeros((L,), jnp.float32)
    for b in (0, 2):
      @pl.loop(0, CH)
      def _(r):
        for j in range(D // L):
          rows[b][r, pl.ds(j * L, L)] = zv

    base = s * ROWS_PER_TILE
    nzf = ROWS_PER_TILE // CH
    nzr = ROWS_PER_TILE - nzf * CH
    for k in range(nzf):
      pltpu.sync_copy(rows[0], acc_sh.at[pl.ds(base + k * CH, CH)])
    if nzr:
      pltpu.sync_copy(rows[0].at[pl.ds(0, nzr)],
                      acc_sh.at[pl.ds(base + nzf * CH, nzr)])
    plsc.subcore_barrier()

    e_wait(0, 0)
    pltpu.async_copy(rows[2], acc_sh.at[ed[0].at[1]], ssem[2], add=True)
    g_issue(0, 0)
    e_wait(1, 1)
    g_issue(1, 1)

    @pl.loop(0, NCH // 6)
    def _(k):
      c0 = k * 6
      for j in range(6):
        c = c0 + j
        j3 = j % 3
        g_wait(j, j3)
        s_wait((j + 5) % 6, (j + 2) % 3)
        cg = lax.rem(c + 2, NCH)
        e_wait(cg, (j + 2) % 6)
        g_issue((j + 2) % 6, (j + 2) % 3)
        ce = lax.rem(c + 4, NCH)
        e_issue(ce, (j + 4) % 6)
        scale(j, j3)
        s_issue(j, j3)

    for t in range(2):
      e_wait(t + 2, t + 2)
    for t in range(2):
      g_wait(t, t)
    s_wait(5, 2)

    plsc.subcore_barrier()

    for k in range(nzf):
      sl = pl.ds(base + k * CH, CH)
      pltpu.sync_copy(acc_sh.at[sl], out_h.at[ci, sl])
    if nzr:
      sl = pl.ds(base + nzf * CH, nzr)
      pltpu.sync_copy(acc_sh.at[sl], out_h.at[ci, sl])

  return spmm


_spmm = _make_spmm()


def _make_grouppool():
  @functools.partial(
      pl.kernel,
      out_type=(
          jax.ShapeDtypeStruct((GP, D), jnp.float32),
          jax.ShapeDtypeStruct((GP, D), jnp.float32),
      ),
      mesh=_mesh,
      scratch_types=[
          pltpu.VMEM((GCH, MCH), jnp.int32),
          pltpu.VMEM((MCH, D), jnp.float32),
          pltpu.VMEM((MCH // GS, D), jnp.float32),
          pltpu.VMEM((MCH // GS, D), jnp.float32),
          pltpu.SemaphoreType.DMA,
      ],
  )
  def grouppool(emb_hbm, memb_h, mid_h, half_h,
                midx_v, rows_v, mid_v, half_v, gsem):
    ci = lax.axis_index("c")
    s = lax.axis_index("s")
    wid = ci * NS + s
    gpc = MCH // GS

    pltpu.sync_copy(memb_h.at[wid], midx_v)
    for k in range(GCH):
      pltpu.async_copy(emb_hbm.at[midx_v.at[k]], rows_v, gsem).wait()

      @pl.loop(0, gpc)
      def _(g):
        r0 = g * GS
        for j in range(D // L):
          sl = pl.ds(j * L, L)
          mx = rows_v[r0, sl]
          mn = mx
          for m in range(1, GS):
            r = rows_v[r0 + m, sl]
            mx = jnp.maximum(mx, r)
            mn = jnp.minimum(mn, r)
          mid_v[g, sl] = (mx + mn) * 0.5
          half_v[g, sl] = (mx - mn) * 0.5

      obase = wid * (GCH * gpc) + k * gpc
      pltpu.sync_copy(mid_v, mid_h.at[pl.ds(obase, gpc)])
      pltpu.sync_copy(half_v, half_h.at[pl.ds(obase, gpc)])

  return grouppool


_grouppool = _make_grouppool()


def _combine(p01, accin):
  def body(p_ref, acc_ref, x_ref, accout_ref):
    xv = p_ref[0] + p_ref[1]
    x_ref[...] = xv
    accout_ref[...] = acc_ref[...] + xv

  nb = 10
  rb = N // nb
  return pl.pallas_call(
      body,
      grid=(nb,),
      in_specs=[
          pl.BlockSpec((2, rb, D), lambda i: (0, i, 0)),
          pl.BlockSpec((rb, D), lambda i: (i, 0)),
      ],
      out_specs=[
          pl.BlockSpec((rb, D), lambda i: (i, 0)),
          pl.BlockSpec((rb, D), lambda i: (i, 0)),
      ],
      out_shape=(
          jax.ShapeDtypeStruct((N, D), jnp.float32),
          jax.ShapeDtypeStruct((N, D), jnp.float32),
      ),
  )(p01, accin)


def _finalize(p01, accin):
  def body(p_ref, acc_ref, emb_ref):
    emb_ref[...] = (acc_ref[...] + p_ref[0] + p_ref[1]) * 0.25

  nb = 10
  rb = N // nb
  return pl.pallas_call(
      body,
      grid=(nb,),
      in_specs=[
          pl.BlockSpec((2, rb, D), lambda i: (0, i, 0)),
          pl.BlockSpec((rb, D), lambda i: (i, 0)),
      ],
      out_specs=pl.BlockSpec((rb, D), lambda i: (i, 0)),
      out_shape=jax.ShapeDtypeStruct((N, D), jnp.float32),
  )(p01, accin)


def _groupmm(mid, half, wc_w, wo_w):
  def body(m_ref, h_ref, wc_ref, wo_ref, c_ref, o_ref):
    dn = (((1,), (1,)), ((), ()))
    c_ref[...] = lax.dot_general(
        m_ref[...], wc_ref[...], dn,
        precision=lax.Precision.HIGHEST,
        preferred_element_type=jnp.float32)
    o_ref[...] = lax.dot_general(
        h_ref[...], wo_ref[...], dn,
        precision=lax.Precision.HIGHEST,
        preferred_element_type=jnp.float32)

  return pl.pallas_call(
      body,
      out_shape=(
          jax.ShapeDtypeStruct((GP, D), jnp.float32),
          jax.ShapeDtypeStruct((GP, D), jnp.float32),
      ),
  )(mid, half, wc_w, wo_w)


@jax.jit
def kernel(edge_index, edge_values, members, user_table, item_table, wc_w, wo_w):
  x0 = jnp.concatenate([user_table, item_table], axis=0)

  dst = edge_index[0]
  src = edge_index[1]
  epad = NW * EW - E
  spread = (jnp.arange(epad, dtype=jnp.int32) * 37) % N
  srcp = jnp.concatenate([src, spread]).reshape(NW, NCH, CH)
  dstp = jnp.concatenate([dst, spread]).reshape(NW, NCH, CH)
  valb = lax.bitcast_convert_type(
      jnp.concatenate([edge_values, jnp.zeros((epad,), jnp.float32)]),
      jnp.int32).reshape(NW, NCH, CH)
  edata = jnp.stack([srcp, dstp, valb], axis=2)

  x = x0
  acc = x0
  for layer in range(N_LAYERS):
    p01 = _spmm(x, edata)
    if layer < N_LAYERS - 1:
      x, acc = _combine(p01, acc)
    else:
      emb = _finalize(p01, acc)

  mpad = GP * GS - G * GS
  membp = jnp.concatenate(
      [members.reshape(-1), jnp.zeros((mpad,), jnp.int32)]).reshape(NW, GCH, MCH)
  mid, half = _grouppool(emb, membp)
  centers, offsets = _groupmm(mid, half, wc_w, wo_w)

  return (emb[:NUM_USERS], emb[NUM_USERS:], centers[:G], offsets[:G])


# speedup vs baseline: 1.0092x; 1.0092x over previous
import dataclasses
import functools

import jax
import jax.numpy as jnp
from jax import lax
from jax.experimental import pallas as pl
from jax.experimental.pallas import tpu as pltpu
from jax.experimental.pallas import tpu_sc as plsc

NUM_USERS = 5000
NUM_ITEMS = 5000
N = NUM_USERS + NUM_ITEMS
E = 320000
D = 128
N_LAYERS = 3
G = 1000
GS = 8

NC = 2
NS = 16
NW = NC * NS
L = 16

CH = 112
NCH = 90
EW = NCH * CH
NP = 10240
ROWS_PER_TILE = NP // NS

GP = NW * 32
MCH = 128
GCH = 2

_mesh = plsc.VectorSubcoreMesh(
    core_axis_name="c", subcore_axis_name="s", num_cores=NC, num_subcores=NS
)

_sc_params = pltpu.CompilerParams()
if "needs_layout_passes" in pltpu.CompilerParams.__dataclass_fields__:
  _sc_params = dataclasses.replace(_sc_params, needs_layout_passes=False)


def _make_spmm():
  rows_t = [pltpu.VMEM((CH, D), jnp.float32) for _ in range(3)]
  ed_t = [pltpu.VMEM((3, CH), jnp.int32) for _ in range(6)]
  sems = [pltpu.SemaphoreType.DMA for _ in range(12)]

  @functools.partial(
      pl.kernel,
      out_type=jax.ShapeDtypeStruct((NC, NP, D), jnp.float32),
      mesh=_mesh,
      scratch_types=(
          rows_t + ed_t + [pltpu.VMEM_SHARED((NP, D), jnp.float32)] + sems
      ),
      compiler_params=_sc_params,
  )
  def spmm(x_hbm, edata_h, out_h, *scr):
    rows = list(scr[0:3])
    ed = list(scr[3:9])
    acc_sh = scr[9]
    gsem = list(scr[10:13])
    ssem = list(scr[13:16])
    esem = list(scr[16:22])

    ci = lax.axis_index("c")
    s = lax.axis_index("s")
    wid = ci * NS + s

    def e_issue(c, j8):
      pltpu.async_copy(edata_h.at[wid, c], ed[j8], esem[j8])

    def e_wait(c, j8):
      pltpu.make_async_copy(edata_h.at[wid, c], ed[j8], esem[j8]).wait()

    def g_issue(j8, j4):
      pltpu.async_copy(x_hbm.at[ed[j8].at[0]], rows[j4], gsem[j4])

    def g_wait(j8, j4):
      pltpu.make_async_copy(x_hbm.at[ed[j8].at[0]], rows[j4], gsem[j4]).wait()

    def s_issue(j8, j4):
      pltpu.async_copy(rows[j4], acc_sh.at[ed[j8].at[1]], ssem[j4], add=True)

    def s_wait(j8, j4):
      pltpu.make_async_copy(rows[j4], acc_sh.at[ed[j8].at[1]], ssem[j4]).wait()

    def scale(j8, j4):
      rows_v = rows[j4]
      ed_v = ed[j8]

      @pl.loop(0, CH, step=L)
      def _(e0):
        v16 = plsc.bitcast(ed_v[2, pl.ds(e0, L)], jnp.float32)
        for i in range(L):
          bv = jnp.full((L,), v16[i], jnp.float32)
          for j in range(D // L):
            sl = pl.ds(j * L, L)
            rows_v[e0 + i, sl] = rows_v[e0 + i, sl] * bv

    for c in range(4):
      e_issue(c, c)

    zv = jnp.zeros((L,), jnp.float32)
    for b in (0, 2):
      @pl.loop(0, CH)
      def _(r):
        for j in range(D // L):
          rows[b][r, pl.ds(j * L, L)] = zv

    base = s * ROWS_PER_TILE
    nzf = ROWS_PER_TILE // CH
    nzr = ROWS_PER_TILE - nzf * CH
    for k in range(nzf):
      pltpu.sync_copy(rows[0], acc_sh.at[pl.ds(base + k * CH, CH)])
    if nzr:
      pltpu.sync_copy(rows[0].at[pl.ds(0, nzr)],
                      acc_sh.at[pl.ds(base + nzf * CH, nzr)])
    plsc.subcore_barrier()

    e_wait(0, 0)
    pltpu.async_copy(rows[2], acc_sh.at[ed[0].at[1]], ssem[2], add=True)
    g_issue(0, 0)
    e_wait(1, 1)
    g_issue(1, 1)

    @pl.loop(0, NCH // 6)
    def _(k):
      c0 = k * 6
      for j in range(6):
        c = c0 + j
        j3 = j % 3
        g_wait(j, j3)
        scale(j, j3)
        s_issue(j, j3)
        s_wait((j + 5) % 6, (j + 2) % 3)
        cg = lax.rem(c + 2, NCH)
        e_wait(cg, (j + 2) % 6)
        g_issue((j + 2) % 6, (j + 2) % 3)
        ce = lax.rem(c + 4, NCH)
        e_issue(ce, (j + 4) % 6)

    for t in range(2):
      e_wait(t + 2, t + 2)
    for t in range(2):
      g_wait(t, t)
    s_wait(5, 2)

    plsc.subcore_barrier()

    for k in range(nzf):
      sl = pl.ds(base + k * CH, CH)
      pltpu.sync_copy(acc_sh.at[sl], out_h.at[ci, sl])
    if nzr:
      sl = pl.ds(base + nzf * CH, nzr)
      pltpu.sync_copy(acc_sh.at[sl], out_h.at[ci, sl])

  return spmm


_spmm = _make_spmm()


def _make_grouppool():
  @functools.partial(
      pl.kernel,
      out_type=(
          jax.ShapeDtypeStruct((GP, D), jnp.float32),
          jax.ShapeDtypeStruct((GP, D), jnp.float32),
      ),
      mesh=_mesh,
      scratch_types=[
          pltpu.VMEM((GCH, MCH), jnp.int32),
          pltpu.VMEM((MCH, D), jnp.float32),
          pltpu.VMEM((MCH // GS, D), jnp.float32),
          pltpu.VMEM((MCH // GS, D), jnp.float32),
          pltpu.SemaphoreType.DMA,
      ],
  )
  def grouppool(emb_hbm, memb_h, mid_h, half_h,
                midx_v, rows_v, mid_v, half_v, gsem):
    ci = lax.axis_index("c")
    s = lax.axis_index("s")
    wid = ci * NS + s
    gpc = MCH // GS

    pltpu.sync_copy(memb_h.at[wid], midx_v)
    for k in range(GCH):
      pltpu.async_copy(emb_hbm.at[midx_v.at[k]], rows_v, gsem).wait()

      @pl.loop(0, gpc)
      def _(g):
        r0 = g * GS
        for j in range(D // L):
          sl = pl.ds(j * L, L)
          mx = rows_v[r0, sl]
          mn = mx
          for m in range(1, GS):
            r = rows_v[r0 + m, sl]
            mx = jnp.maximum(mx, r)
            mn = jnp.minimum(mn, r)
          mid_v[g, sl] = (mx + mn) * 0.5
          half_v[g, sl] = (mx - mn) * 0.5

      obase = wid * (GCH * gpc) + k * gpc
      pltpu.sync_copy(mid_v, mid_h.at[pl.ds(obase, gpc)])
      pltpu.sync_copy(half_v, half_h.at[pl.ds(obase, gpc)])

  return grouppool


_grouppool = _make_grouppool()


def _combine(p01, accin):
  def body(p_ref, acc_ref, x_ref, accout_ref):
    xv = p_ref[0] + p_ref[1]
    x_ref[...] = xv
    accout_ref[...] = acc_ref[...] + xv

  nb = 10
  rb = N // nb
  return pl.pallas_call(
      body,
      grid=(nb,),
      in_specs=[
          pl.BlockSpec((2, rb, D), lambda i: (0, i, 0)),
          pl.BlockSpec((rb, D), lambda i: (i, 0)),
      ],
      out_specs=[
          pl.BlockSpec((rb, D), lambda i: (i, 0)),
          pl.BlockSpec((rb, D), lambda i: (i, 0)),
      ],
      out_shape=(
          jax.ShapeDtypeStruct((N, D), jnp.float32),
          jax.ShapeDtypeStruct((N, D), jnp.float32),
      ),
  )(p01, accin)


def _finalize(p01, accin):
  def body(p_ref, acc_ref, emb_ref):
    emb_ref[...] = (acc_ref[...] + p_ref[0] + p_ref[1]) * 0.25

  nb = 10
  rb = N // nb
  return pl.pallas_call(
      body,
      grid=(nb,),
      in_specs=[
          pl.BlockSpec((2, rb, D), lambda i: (0, i, 0)),
          pl.BlockSpec((rb, D), lambda i: (i, 0)),
      ],
      out_specs=pl.BlockSpec((rb, D), lambda i: (i, 0)),
      out_shape=jax.ShapeDtypeStruct((N, D), jnp.float32),
  )(p01, accin)


def _groupmm(mid, half, wc_w, wo_w):
  def body(m_ref, h_ref, wc_ref, wo_ref, c_ref, o_ref):
    dn = (((1,), (1,)), ((), ()))
    c_ref[...] = lax.dot_general(
        m_ref[...], wc_ref[...], dn,
        precision=lax.Precision.HIGHEST,
        preferred_element_type=jnp.float32)
    o_ref[...] = lax.dot_general(
        h_ref[...], wo_ref[...], dn,
        precision=lax.Precision.HIGHEST,
        preferred_element_type=jnp.float32)

  return pl.pallas_call(
      body,
      out_shape=(
          jax.ShapeDtypeStruct((GP, D), jnp.float32),
          jax.ShapeDtypeStruct((GP, D), jnp.float32),
      ),
  )(mid, half, wc_w, wo_w)


@jax.jit
def kernel(edge_index, edge_values, members, user_table, item_table, wc_w, wo_w):
  x0 = jnp.concatenate([user_table, item_table], axis=0)

  dst = edge_index[0]
  src = edge_index[1]
  epad = NW * EW - E
  spread = (jnp.arange(epad, dtype=jnp.int32) * 37) % N
  srcp = jnp.concatenate([src, spread]).reshape(NW, NCH, CH)
  dstp = jnp.concatenate([dst, spread]).reshape(NW, NCH, CH)
  valb = lax.bitcast_convert_type(
      jnp.concatenate([edge_values, jnp.zeros((epad,), jnp.float32)]),
      jnp.int32).reshape(NW, NCH, CH)
  edata = jnp.stack([srcp, dstp, valb], axis=2)

  x = x0
  acc = x0
  for layer in range(N_LAYERS):
    p01 = _spmm(x, edata)
    if layer < N_LAYERS - 1:
      x, acc = _combine(p01, acc)
    else:
      emb = _finalize(p01, acc)

  mpad = GP * GS - G * GS
  membp = jnp.concatenate(
      [members.reshape(-1), jnp.zeros((mpad,), jnp.int32)]).reshape(NW, GCH, MCH)
  mid, half = _grouppool(emb, membp)
  centers, offsets = _groupmm(mid, half, wc_w, wo_w)

  return (emb[:NUM_USERS], emb[NUM_USERS:], centers[:G], offsets[:G])
